# baseline (speedup 1.0000x reference)
import functools

import jax
import jax.numpy as jnp
from jax import lax
from jax.experimental import pallas as pl
from jax.experimental.pallas import tpu as pltpu
from jax.experimental.pallas import tpu_sc as plsc

D_MODEL = 1024
MAX_LEN = 128
VISIBLE_RANGE = 9.0
NSIDE = 10
NROWS = 112

NC, NS, L = 2, 16, 16
NW = NC * NS

B_TOTAL = 4 * 2048
B_PER_W = B_TOTAL // NW
CHUNK = 16
N_CHUNK = B_PER_W // CHUNK
NBUF = 4

_MAGIC = 2.0**23


def _round_half_even(v):
    return ((v + _MAGIC) - _MAGIC).astype(jnp.int32)


def _sc_gather(tokens_flat, enc_flat):
    mesh = plsc.VectorSubcoreMesh(core_axis_name="c", subcore_axis_name="s")

    @functools.partial(
        pl.kernel,
        mesh=mesh,
        out_type=jax.ShapeDtypeStruct((B_TOTAL * D_MODEL,), jnp.float32),
        scratch_types=[
            pltpu.VMEM((B_PER_W * 2,), jnp.float32),
            pltpu.VMEM((B_PER_W,), jnp.int32),
            pltpu.VMEM((CHUNK, D_MODEL), jnp.float32),
        ]
        + [pltpu.VMEM((CHUNK * D_MODEL,), jnp.float32) for _ in range(NBUF)]
        + [pltpu.VMEM_SHARED((NROWS, D_MODEL), jnp.float32)]
        + [pltpu.SemaphoreType.DMA for _ in range(2 * NBUF + 1)],
    )
    def k(tok_hbm, enc_hbm, out_hbm, tok_v, idx_v, stage_v,
          b0, b1, b2, b3, table_s, g0, g1, g2, g3, w0, w1, w2, w3, ssem):
        bufs = (b0, b1, b2, b3)
        gsems = (g0, g1, g2, g3)
        wsems = (w0, w1, w2, w3)

        sid = lax.axis_index("s")
        wid = sid * NC + lax.axis_index("c")
        base = wid * B_PER_W

        pltpu.sync_copy(tok_hbm.at[pl.ds(base, B_PER_W)],
                        tok_v.at[pl.ds(0, B_PER_W)])
        pltpu.sync_copy(tok_hbm.at[pl.ds(B_TOTAL + base, B_PER_W)],
                        tok_v.at[pl.ds(B_PER_W, B_PER_W)])

        for i in range(B_PER_W // L):
            x = tok_v[pl.ds(i * L, L)]
            y = tok_v[pl.ds(B_PER_W + i * L, L)]
            rx = _round_half_even(x * VISIBLE_RANGE)
            ry = _round_half_even(y * VISIBLE_RANGE)
            idx_v[pl.ds(i * L, L)] = rx * NSIDE + ry

        @pl.when(sid == 0)
        def _stage():
            lanes = lax.iota(jnp.int32, L)
            for c in range(NROWS // L):
                kk = jnp.minimum(lanes + (c * L), NSIDE * NSIDE - 1)
                fidx = lax.div(kk, NSIDE) * MAX_LEN + lax.rem(kk, NSIDE)
                pltpu.async_copy(enc_hbm.at[fidx], stage_v, ssem).wait()
                pltpu.sync_copy(stage_v, table_s.at[pl.ds(c * L, L)])

        plsc.subcore_barrier()

        def gather_chunk(j):
            b = j % NBUF
            handles = []
            rows_vec = idx_v[pl.ds(j * CHUNK, CHUNK)]
            for t in range(CHUNK):
                row = rows_vec[t]
                handles.append(pltpu.async_copy(
                    table_s.at[row],
                    bufs[b].at[pl.ds(t * D_MODEL, D_MODEL)],
                    gsems[b]))
            return handles

        def drain_and_write(j, handles):
            b = j % NBUF
            for h in handles:
                h.wait()
            return pltpu.async_copy(
                bufs[b],
                out_hbm.at[pl.ds((base + j * CHUNK) * D_MODEL,
                                 CHUNK * D_MODEL)],
                wsems[b])

        writes = [None] * NBUF
        gathers = [None] * N_CHUNK
        gathers[0] = gather_chunk(0)
        for j in range(N_CHUNK):
            if j + 1 < N_CHUNK:
                b = (j + 1) % NBUF
                if writes[b] is not None:
                    writes[b].wait()
                    writes[b] = None
                gathers[j + 1] = gather_chunk(j + 1)
            writes[j % NBUF] = drain_and_write(j, gathers[j])
        for wr in writes:
            if wr is not None:
                wr.wait()

    return k(tokens_flat, enc_flat)


def kernel(tokens, encoding):
    b, s, _ = tokens.shape
    tokens_flat = tokens.reshape(b * s, 2).T.reshape(b * s * 2)
    enc_flat = encoding.reshape(MAX_LEN * MAX_LEN, D_MODEL)
    out = _sc_gather(tokens_flat, enc_flat)
    return out.reshape(b, s, D_MODEL)
